# baseline (speedup 1.0000x reference)
import functools

import jax
import jax.numpy as jnp
from jax import lax
from jax.experimental import pallas as pl
from jax.experimental.pallas import tpu as pltpu
from jax.experimental.pallas import tpu_sc as plsc

N_EXP = 8
BLOCK = 4096
N_WORKERS = 32
LANES = 16


def _scores_kernel(x_ref, w_ref, b_ref, s_ref):
    x = x_ref[...]
    w = w_ref[...]
    s = jax.lax.dot_general(
        w, x, (((1,), (1,)), ((), ())),
        preferred_element_type=jnp.float32,
    )
    s_ref[...] = s + b_ref[...][:, None]


def _routing_kernel(tok_per_w, s_hbm, out_hbm, s_v, out_v):
    wid = lax.axis_index("s") * 2 + lax.axis_index("c")
    base = wid * tok_per_w
    pltpu.sync_copy(s_hbm.at[:, pl.ds(base, tok_per_w)], s_v)

    lane = lax.iota(jnp.int32, LANES)
    zeros = jnp.zeros((LANES,), jnp.float32)

    def body(t, carry):
        svec = [s_v[e, pl.ds(t * LANES, LANES)] for e in range(N_EXP)]
        m = svec[0]
        for e in range(1, N_EXP):
            m = jnp.maximum(m, svec[e])
        denom = jnp.exp(svec[0] - m)
        amax = jnp.where(svec[0] == m, 0, N_EXP)
        for e in range(1, N_EXP):
            denom = denom + jnp.exp(svec[e] - m)
            amax = jnp.minimum(amax, jnp.where(svec[e] == m, e, N_EXP))
        val = 1.0 / denom
        for j in range(N_EXP):
            out_v[pl.ds(t * LANES * N_EXP + j * LANES, LANES)] = zeros
        idx = t * (LANES * N_EXP) + lane * N_EXP + amax
        plsc.store_scatter(out_v, [idx], val)
        return carry

    lax.fori_loop(0, tok_per_w // LANES, body, 0)
    pltpu.sync_copy(out_v, out_hbm.at[pl.ds(base * N_EXP, tok_per_w * N_EXP)])


def kernel(x, W, b):
    n_tokens, c_in = x.shape
    scores_t = pl.pallas_call(
        _scores_kernel,
        grid=(n_tokens // BLOCK,),
        in_specs=[
            pl.BlockSpec((BLOCK, c_in), lambda i: (i, 0)),
            pl.BlockSpec((N_EXP, c_in), lambda i: (0, 0)),
            pl.BlockSpec((N_EXP,), lambda i: (0,)),
        ],
        out_specs=pl.BlockSpec((N_EXP, BLOCK), lambda i: (0, i)),
        out_shape=jax.ShapeDtypeStruct((N_EXP, n_tokens), jnp.float32),
    )(x, W, b)

    tok_per_w = n_tokens // N_WORKERS
    mesh = plsc.VectorSubcoreMesh(core_axis_name="c", subcore_axis_name="s")
    routing = functools.partial(
        pl.kernel,
        mesh=mesh,
        compiler_params=pltpu.CompilerParams(needs_layout_passes=False),
        out_type=jax.ShapeDtypeStruct((n_tokens * N_EXP,), jnp.float32),
        scratch_types=[
            pltpu.VMEM((N_EXP, tok_per_w), jnp.float32),
            pltpu.VMEM((tok_per_w * N_EXP,), jnp.float32),
        ],
    )(functools.partial(_routing_kernel, tok_per_w))
    flat = routing(scores_t)
    return flat.reshape(n_tokens, N_EXP)


# speedup vs baseline: 1.4948x; 1.4948x over previous
import jax
import jax.numpy as jnp
from jax.experimental import pallas as pl

N_EXP = 8
BLOCK = 4096


def _gate_kernel(x_ref, w_ref, b_ref, o_ref):
    x = x_ref[...]
    w = w_ref[...]
    s = jax.lax.dot_general(
        x, w, (((1,), (1,)), ((), ())),
        preferred_element_type=jnp.float32,
    ) + b_ref[...][None, :]
    m = jnp.max(s, axis=1, keepdims=True)
    e = jnp.exp(s - m)
    denom = jnp.sum(e, axis=1, keepdims=True)
    iota = jax.lax.broadcasted_iota(jnp.int32, s.shape, 1)
    amax = jnp.min(jnp.where(s == m, iota, N_EXP), axis=1, keepdims=True)
    o_ref[...] = jnp.where(iota == amax, 1.0 / denom, 0.0)


def kernel(x, W, b):
    n_tokens, c_in = x.shape
    grid = (n_tokens // BLOCK,)
    return pl.pallas_call(
        _gate_kernel,
        grid=grid,
        in_specs=[
            pl.BlockSpec((BLOCK, c_in), lambda i: (i, 0)),
            pl.BlockSpec((N_EXP, c_in), lambda i: (0, 0)),
            pl.BlockSpec((N_EXP,), lambda i: (0,)),
        ],
        out_specs=pl.BlockSpec((BLOCK, N_EXP), lambda i: (i, 0)),
        out_shape=jax.ShapeDtypeStruct((n_tokens, N_EXP), jnp.float32),
    )(x, W, b)
